# Initial kernel scaffold; baseline (speedup 1.0000x reference)
import jax
import jax.numpy as jnp
from jax.experimental import pallas as pl


def kernel(ref_positions, query_positions):
    raise NotImplementedError("write your pallas kernel here")


# speedup vs baseline: 4.0778x; 4.0778x over previous
import functools

import jax
import jax.numpy as jnp
from jax import lax
from jax.experimental import pallas as pl
from jax.experimental.pallas import tpu as pltpu
from jax.experimental.pallas import tpu_sc as plsc

N_REF = 8192
N_QUERY = 2048
RADIUS2 = 0.25

BM = 128


def _mask_kernel(q_ref, rt_ref, mask_ref, mi32_ref, cnt_ref):
    q = q_ref[...]
    rt = rt_ref[...]
    dot = jnp.dot(q, rt, precision=lax.Precision.DEFAULT)
    q_sq = jnp.sum(q * q, axis=1, keepdims=True)
    r_sq = jnp.sum(rt * rt, axis=0, keepdims=True)
    dist2 = q_sq + r_sq - 2.0 * dot
    dist2 = jnp.maximum(dist2, 0.0)
    mask = dist2 <= RADIUS2
    mask_ref[...] = mask
    mi32 = mask.astype(jnp.int32)
    mi32_ref[...] = mi32
    cnt_ref[...] = jnp.sum(mi32, axis=1, keepdims=True)


def _offsets_kernel(cnt_ref, out_ref):
    cnt = cnt_ref[...].astype(jnp.float32)
    k = lax.broadcasted_iota(jnp.int32, (128, 128), 0)
    l = lax.broadcasted_iota(jnp.int32, (128, 128), 1)
    tri = (k <= l).astype(jnp.float32)
    incl = jnp.dot(cnt, tri, precision=lax.Precision.HIGHEST)
    i = lax.broadcasted_iota(jnp.int32, (16, 16), 0)
    j = lax.broadcasted_iota(jnp.int32, (16, 16), 1)
    strict = (j < i).astype(jnp.float32)
    row_tot = incl[:, 127:128]
    row_off = jnp.dot(strict, row_tot, precision=lax.Precision.HIGHEST)
    excl = row_off + incl - cnt
    out_ref[0:16, :] = excl.astype(jnp.int32)
    total = row_off[15:16, 0:1] + incl[15:16, 127:128]
    out_ref[16:17, :] = jnp.broadcast_to(total, (1, 128)).astype(jnp.int32)


def _sc_body(mask_hbm, out_hbm, mrow, rowbuf, sem):
    info = plsc.get_sparse_core_info()
    nc = info.num_cores
    wid = lax.axis_index("s") * nc + lax.axis_index("c")
    nw = nc * info.num_subcores
    rows_per_w = N_QUERY // nw
    chunks = N_REF // 16
    lanes = lax.iota(jnp.int32, 16)
    zeros16 = jnp.zeros((16,), jnp.int32)

    def do_row(r, carry):
        row = wid * rows_per_w + r
        pltpu.sync_copy(mask_hbm.at[row], mrow)

        def fwd(c, trun):
            m = mrow[pl.ds(c * 16, 16)]
            mb = m > 0
            incl = plsc.cumsum(m)
            jv = c * 16 + lanes
            pos = trun + incl - 1
            plsc.store_scatter(rowbuf, [pos], jv, mask=mb)
            return trun + plsc.all_reduce_population_count(mb)

        lax.fori_loop(0, chunks, fwd, zeros16, unroll=2)

        def bwd(c2, frun):
            c = (chunks - 1) - c2
            m = mrow[pl.ds(c * 16, 16)]
            fb = m == 0
            incl_f = plsc.cumsum(1 - m)
            nf = plsc.all_reduce_population_count(fb)
            jv = c * 16 + lanes
            pos = (N_REF - 1) - frun - nf + incl_f
            plsc.store_scatter(rowbuf, [pos], jv, mask=fb)
            return frun + nf

        lax.fori_loop(0, chunks, bwd, zeros16, unroll=2)
        pltpu.sync_copy(rowbuf.at[pl.ds(0, N_REF)], out_hbm.at[row])
        return carry

    lax.fori_loop(0, rows_per_w, do_row, 0)


@functools.partial(jax.jit, static_argnums=())
def kernel(ref_positions, query_positions):
    f32 = jnp.float32
    q8 = jnp.concatenate(
        [query_positions, jnp.zeros((N_QUERY, 5), f32)], axis=1)
    rt8 = jnp.concatenate(
        [ref_positions.T, jnp.zeros((5, N_REF), f32)], axis=0)

    grid = N_QUERY // BM
    mask, mi32, counts2d = pl.pallas_call(
        _mask_kernel,
        grid=(grid,),
        in_specs=[
            pl.BlockSpec((BM, 8), lambda i: (i, 0)),
            pl.BlockSpec((8, N_REF), lambda i: (0, 0)),
        ],
        out_specs=[
            pl.BlockSpec((BM, N_REF), lambda i: (i, 0)),
            pl.BlockSpec((BM, N_REF), lambda i: (i, 0)),
            pl.BlockSpec((BM, 1), lambda i: (i, 0)),
        ],
        out_shape=[
            jax.ShapeDtypeStruct((N_QUERY, N_REF), jnp.bool_),
            jax.ShapeDtypeStruct((N_QUERY, N_REF), jnp.int32),
            jax.ShapeDtypeStruct((N_QUERY, 1), jnp.int32),
        ],
    )(q8, rt8)

    offsets_2d = pl.pallas_call(
        _offsets_kernel,
        out_shape=jax.ShapeDtypeStruct((17, 128), jnp.int32),
    )(counts2d.reshape(16, 128))
    offsets = offsets_2d.reshape(-1)[: N_QUERY + 1]

    mesh = plsc.VectorSubcoreMesh(core_axis_name="c", subcore_axis_name="s")
    neighbor_index = pl.kernel(
        _sc_body,
        out_type=jax.ShapeDtypeStruct((N_QUERY, N_REF), jnp.int32),
        mesh=mesh,
        compiler_params=pltpu.CompilerParams(needs_layout_passes=False),
        scratch_types=[
            pltpu.VMEM((N_REF,), jnp.int32),
            pltpu.VMEM((N_REF + 16,), jnp.int32),
            pltpu.SemaphoreType.DMA,
        ],
    )(mi32)

    return neighbor_index, counts2d.reshape(N_QUERY), offsets, mask


# speedup vs baseline: 22.5121x; 5.5206x over previous
import functools

import jax
import jax.numpy as jnp
from jax import lax
from jax.experimental import pallas as pl
from jax.experimental.pallas import tpu as pltpu
from jax.experimental.pallas import tpu_sc as plsc

N_REF = 8192
N_QUERY = 2048
RADIUS2 = 0.25

BM = 128


def _mask_kernel(q_ref, rt_ref, mask_ref, mi32_ref, cnt_ref):
    q = q_ref[...]
    rt = rt_ref[...]
    dot = jnp.dot(q, rt, precision=lax.Precision.DEFAULT)
    q_sq = jnp.sum(q * q, axis=1, keepdims=True)
    r_sq = jnp.sum(rt * rt, axis=0, keepdims=True)
    dist2 = q_sq + r_sq - 2.0 * dot
    dist2 = jnp.maximum(dist2, 0.0)
    mask = dist2 <= RADIUS2
    mask_ref[...] = mask
    mi32 = mask.astype(jnp.int32)
    mi32_ref[...] = mi32
    cnt_ref[...] = jnp.sum(mi32, axis=1, keepdims=True)


def _offsets_kernel(cnt_ref, out_ref):
    cnt = cnt_ref[...].astype(jnp.float32)
    k = lax.broadcasted_iota(jnp.int32, (128, 128), 0)
    l = lax.broadcasted_iota(jnp.int32, (128, 128), 1)
    tri = (k <= l).astype(jnp.float32)
    incl = jnp.dot(cnt, tri, precision=lax.Precision.HIGHEST)
    i = lax.broadcasted_iota(jnp.int32, (16, 16), 0)
    j = lax.broadcasted_iota(jnp.int32, (16, 16), 1)
    strict = (j < i).astype(jnp.float32)
    row_tot = incl[:, 127:128]
    row_off = jnp.dot(strict, row_tot, precision=lax.Precision.HIGHEST)
    excl = row_off + incl - cnt
    out_ref[0:16, :] = excl.astype(jnp.int32)
    total = row_off[15:16, 0:1] + incl[15:16, 127:128]
    out_ref[16:17, :] = jnp.broadcast_to(total, (1, 128)).astype(jnp.int32)


def _sc_body(mask_hbm, out_hbm, mrow_a, mrow_b, rowbuf_a, rowbuf_b,
             lsem_a, lsem_b, ssem_a, ssem_b):
    info = plsc.get_sparse_core_info()
    nc = info.num_cores
    wid = lax.axis_index("s") * nc + lax.axis_index("c")
    nw = nc * info.num_subcores
    rows_per_w = N_QUERY // nw
    base_row = wid * rows_per_w
    chunks = N_REF // 16
    lanes = lax.iota(jnp.int32, 16)
    ones16 = jnp.ones((16,), jnp.int32)

    def step(r, row, mrow, mrow_n, rowbuf, lsem, lsem_n, ssem):
        pltpu.make_async_copy(mask_hbm.at[row], mrow, lsem).wait()

        @pl.when(r + 1 < rows_per_w)
        def _():
            pltpu.async_copy(mask_hbm.at[row + 1], mrow_n, lsem_n)

        @pl.when(r >= 2)
        def _():
            pltpu.make_async_copy(
                rowbuf.at[pl.ds(0, N_REF)], out_hbm.at[row - 2], ssem).wait()

        @plsc.parallel_loop(0, chunks, 1, unroll=8,
                            carry=(lanes, -ones16))
        def fwd(c, carry_in):
            jv, trunm1 = carry_in
            m = mrow[pl.ds(c * 16, 16)]
            mb = m > 0
            incl = plsc.cumsum(ones16, mask=mb)
            pos = trunm1 + incl
            plsc.store_scatter(rowbuf, [pos], jv, mask=mb)
            pc = plsc.all_reduce_population_count(mb)
            return jv + 16, trunm1 + pc

        @plsc.parallel_loop(0, chunks, 1, unroll=8,
                            carry=((chunks - 1) * 16 + lanes,
                                   jnp.full((16,), N_REF - 1, jnp.int32)))
        def bwd(c2, carry_in):
            jv, fbase = carry_in
            c = (chunks - 1) - c2
            m = mrow[pl.ds(c * 16, 16)]
            fb = m == 0
            incl_f = plsc.cumsum(ones16, mask=fb)
            nf = plsc.all_reduce_population_count(fb)
            nbase = fbase - nf
            pos = nbase + incl_f
            plsc.store_scatter(rowbuf, [pos], jv, mask=fb)
            return jv - 16, nbase

        pltpu.async_copy(
            rowbuf.at[pl.ds(0, N_REF)], out_hbm.at[row], ssem)

    pltpu.async_copy(mask_hbm.at[base_row], mrow_a, lsem_a)

    def do_row(r, carry):
        row = base_row + r
        par = r & 1

        @pl.when(par == 0)
        def _():
            step(r, row, mrow_a, mrow_b, rowbuf_a, lsem_a, lsem_b, ssem_a)

        @pl.when(par == 1)
        def _():
            step(r, row, mrow_b, mrow_a, rowbuf_b, lsem_b, lsem_a, ssem_b)

        return carry

    lax.fori_loop(0, rows_per_w, do_row, 0)

    pltpu.make_async_copy(
        rowbuf_a.at[pl.ds(0, N_REF)],
        out_hbm.at[base_row + rows_per_w - 2], ssem_a).wait()
    pltpu.make_async_copy(
        rowbuf_b.at[pl.ds(0, N_REF)],
        out_hbm.at[base_row + rows_per_w - 1], ssem_b).wait()


@functools.partial(jax.jit, static_argnums=())
def kernel(ref_positions, query_positions):
    f32 = jnp.float32
    q8 = jnp.concatenate(
        [query_positions, jnp.zeros((N_QUERY, 5), f32)], axis=1)
    rt8 = jnp.concatenate(
        [ref_positions.T, jnp.zeros((5, N_REF), f32)], axis=0)

    grid = N_QUERY // BM
    mask, mi32, counts2d = pl.pallas_call(
        _mask_kernel,
        grid=(grid,),
        in_specs=[
            pl.BlockSpec((BM, 8), lambda i: (i, 0)),
            pl.BlockSpec((8, N_REF), lambda i: (0, 0)),
        ],
        out_specs=[
            pl.BlockSpec((BM, N_REF), lambda i: (i, 0)),
            pl.BlockSpec((BM, N_REF), lambda i: (i, 0)),
            pl.BlockSpec((BM, 1), lambda i: (i, 0)),
        ],
        out_shape=[
            jax.ShapeDtypeStruct((N_QUERY, N_REF), jnp.bool_),
            jax.ShapeDtypeStruct((N_QUERY, N_REF), jnp.int32),
            jax.ShapeDtypeStruct((N_QUERY, 1), jnp.int32),
        ],
    )(q8, rt8)

    offsets_2d = pl.pallas_call(
        _offsets_kernel,
        out_shape=jax.ShapeDtypeStruct((17, 128), jnp.int32),
    )(counts2d.reshape(16, 128))
    offsets = offsets_2d.reshape(-1)[: N_QUERY + 1]

    mesh = plsc.VectorSubcoreMesh(core_axis_name="c", subcore_axis_name="s")
    neighbor_index = pl.kernel(
        _sc_body,
        out_type=jax.ShapeDtypeStruct((N_QUERY, N_REF), jnp.int32),
        mesh=mesh,
        compiler_params=pltpu.CompilerParams(needs_layout_passes=False),
        scratch_types=[
            pltpu.VMEM((N_REF,), jnp.int32),
            pltpu.VMEM((N_REF,), jnp.int32),
            pltpu.VMEM((N_REF + 16,), jnp.int32),
            pltpu.VMEM((N_REF + 16,), jnp.int32),
            pltpu.SemaphoreType.DMA,
            pltpu.SemaphoreType.DMA,
            pltpu.SemaphoreType.DMA,
            pltpu.SemaphoreType.DMA,
        ],
    )(mi32)

    return neighbor_index, counts2d.reshape(N_QUERY), offsets, mask
